# baseline (speedup 1.0000x reference)
import functools

import jax
import jax.numpy as jnp
from jax import lax
from jax.experimental import pallas as pl
from jax.experimental.pallas import tpu as pltpu
from jax.experimental.pallas import tpu_sc as plsc

N = 10000
D = 128
E = 320000
G = 64
H = 128
Z = 64
MAXN = 160

NC = 2
NS = 16
NW = NC * NS

CHUNK = 128
CPW = 80
ECP = CPW * NW
EPAD = ECP * CHUNK - E
NROW_TILE = 624
NREM = N - NROW_TILE * NS
NAGG = N + CHUNK


NBUF = 2
PCH = 40


def _sc_scatter_body(h_hbm, src_hbm, dst_hbm, zero_hbm, out_hbm,
                     src_v, dst_v, rows_v, gsems, ssems, agg):
    c = lax.axis_index("c")
    s = lax.axis_index("s")
    w = c * NS + s

    pltpu.sync_copy(zero_hbm.at[pl.ds(0, NROW_TILE)],
                    agg.at[pl.ds(s * NROW_TILE, NROW_TILE)])

    @pl.when(s == 0)
    def _():
        pltpu.sync_copy(zero_hbm.at[pl.ds(0, NREM + NAGG - N)],
                        agg.at[pl.ds(NS * NROW_TILE, NREM + NAGG - N)])

    plsc.subcore_barrier()

    for phase in range(CPW // PCH):
        base = w * CPW + phase * PCH
        pltpu.sync_copy(src_hbm.at[pl.ds(base, PCH)], src_v)
        pltpu.sync_copy(dst_hbm.at[pl.ds(base, PCH)], dst_v)

        for b in range(NBUF):
            pltpu.async_copy(h_hbm.at[src_v.at[b]], rows_v.at[b], gsems.at[b])

        @pl.loop(0, PCH - NBUF, step=NBUF)
        def _(j0):
            for b in range(NBUF):
                pltpu.make_async_copy(h_hbm.at[src_v.at[0]], rows_v.at[b],
                                      gsems.at[b]).wait()
                pltpu.async_copy(rows_v.at[b], agg.at[dst_v.at[j0 + b]],
                                 ssems.at[b], add=True)
            for b in range(NBUF):
                pltpu.make_async_copy(rows_v.at[b], agg.at[dst_v.at[0]],
                                      ssems.at[b]).wait()
                pltpu.async_copy(h_hbm.at[src_v.at[j0 + NBUF + b]],
                                 rows_v.at[b], gsems.at[b])

        for b in range(NBUF):
            pltpu.make_async_copy(h_hbm.at[src_v.at[0]], rows_v.at[b],
                                  gsems.at[b]).wait()
            pltpu.async_copy(rows_v.at[b], agg.at[dst_v.at[PCH - NBUF + b]],
                             ssems.at[b], add=True)
        for b in range(NBUF):
            pltpu.make_async_copy(rows_v.at[b], agg.at[dst_v.at[0]],
                                  ssems.at[b]).wait()

    plsc.subcore_barrier()
    pltpu.sync_copy(agg.at[pl.ds(s * NROW_TILE, NROW_TILE)],
                    out_hbm.at[pl.ds(c * N + s * NROW_TILE, NROW_TILE)])

    @pl.when(s == 0)
    def _():
        pltpu.sync_copy(agg.at[pl.ds(NS * NROW_TILE, NREM)],
                        out_hbm.at[pl.ds(c * N + NS * NROW_TILE, NREM)])


def _sc_scatter(h, src2, dst2, zero_init):
    mesh = plsc.VectorSubcoreMesh(core_axis_name="c", subcore_axis_name="s",
                                  num_cores=NC, num_subcores=NS)
    k = pl.kernel(
        _sc_scatter_body,
        out_type=jax.ShapeDtypeStruct((2 * N, D), jnp.float32),
        mesh=mesh,
        scratch_types=[
            pltpu.VMEM((PCH, CHUNK), jnp.int32),
            pltpu.VMEM((PCH, CHUNK), jnp.int32),
            pltpu.VMEM((NBUF, CHUNK, D), jnp.float32),
            pltpu.SemaphoreType.DMA((NBUF,)),
            pltpu.SemaphoreType.DMA((NBUF,)),
            pltpu.VMEM_SHARED((NAGG, D), jnp.float32),
        ],
    )
    return k(h, src2, dst2, zero_init)


def _mlp_body(h_ref, pa_ref, pb_ref, w1_ref, b1_ref, w2_ref, b2_ref, o_ref):
    x = h_ref[...] + pa_ref[...] + pb_ref[...]
    t = jnp.maximum(
        jnp.dot(x, w1_ref[...], preferred_element_type=jnp.float32) + b1_ref[...], 0.0)
    o_ref[...] = jnp.maximum(
        jnp.dot(t, w2_ref[...], preferred_element_type=jnp.float32) + b2_ref[...], 0.0)


_MLP_R = 1000


def _mlp(h, agg2, w1, b1, w2, b2):
    nb = N // _MLP_R
    return pl.pallas_call(
        _mlp_body,
        grid=(nb,),
        in_specs=[
            pl.BlockSpec((_MLP_R, D), lambda i: (i, 0)),
            pl.BlockSpec((_MLP_R, D), lambda i: (i, 0)),
            pl.BlockSpec((_MLP_R, D), lambda i, _nb=nb: (i + _nb, 0)),
            pl.BlockSpec((D, H), lambda i: (0, 0)),
            pl.BlockSpec((1, H), lambda i: (0, 0)),
            pl.BlockSpec((H, H), lambda i: (0, 0)),
            pl.BlockSpec((1, H), lambda i: (0, 0)),
        ],
        out_specs=pl.BlockSpec((_MLP_R, H), lambda i: (i, 0)),
        out_shape=jax.ShapeDtypeStruct((N, H), jnp.float32),
    )(h, agg2, agg2, w1, b1, w2, b2)


def _pool_body(h_ref, b_ref, sums_ref, cnts_ref):
    @pl.when(pl.program_id(0) == 0)
    def _():
        sums_ref[...] = jnp.zeros_like(sums_ref)
        cnts_ref[...] = jnp.zeros_like(cnts_ref)

    bb = b_ref[0, 0, :]
    onehot = (bb[:, None] == lax.broadcasted_iota(jnp.int32, (1, G), 1)
              ).astype(jnp.float32)
    sums_ref[...] += lax.dot_general(
        onehot, h_ref[...], (((0,), (0,)), ((), ())),
        preferred_element_type=jnp.float32)
    cnt = jnp.sum(onehot, axis=0)
    cnts_ref[...] += jnp.broadcast_to(cnt[:, None], (G, H))


def _pool(h2, batch3):
    nb = N // _MLP_R
    return pl.pallas_call(
        _pool_body,
        grid=(nb,),
        in_specs=[
            pl.BlockSpec((_MLP_R, H), lambda i: (i, 0)),
            pl.BlockSpec((1, 1, _MLP_R), lambda i: (i, 0, 0)),
        ],
        out_specs=[
            pl.BlockSpec((G, H), lambda i: (0, 0)),
            pl.BlockSpec((G, H), lambda i: (0, 0)),
        ],
        out_shape=[
            jax.ShapeDtypeStruct((G, H), jnp.float32),
            jax.ShapeDtypeStruct((G, H), jnp.float32),
        ],
    )(h2, batch3)


def _decode_body(sums_ref, cnts_ref, muw_ref, mub_ref, lvw_ref, lvb_ref,
                 eps_ref, dw1_ref, db1_ref, dw2_ref, db2_ref, hd_ref):
    pooled = sums_ref[...] / jnp.maximum(cnts_ref[...], 1.0)
    mu = jnp.dot(pooled, muw_ref[...], preferred_element_type=jnp.float32) + mub_ref[...]
    lv = jnp.clip(
        jnp.dot(pooled, lvw_ref[...], preferred_element_type=jnp.float32) + lvb_ref[...],
        -5.0, 5.0)
    z = mu + eps_ref[...] * jnp.exp(0.5 * lv)
    t = jnp.maximum(
        jnp.dot(z, dw1_ref[...], preferred_element_type=jnp.float32) + db1_ref[...], 0.0)
    hd_ref[...] = jnp.dot(t, dw2_ref[...], preferred_element_type=jnp.float32) + db2_ref[...]


def _decode(sums, cnts, p, eps):
    return pl.pallas_call(
        _decode_body,
        out_shape=jax.ShapeDtypeStruct((G, H), jnp.float32),
    )(sums, cnts, p['mu_w'], p['mu_b'].reshape(1, Z), p['lv_w'],
      p['lv_b'].reshape(1, Z), eps, p['dec_w1'], p['dec_b1'].reshape(1, H),
      p['dec_w2'], p['dec_b2'].reshape(1, H))


_NODE_NB = 8
_NODE_COLS = _NODE_NB * D


def _node_body(hd_ref, nw_ref, nb_ref, o_ref):
    t = jnp.dot(hd_ref[...], nw_ref[...], preferred_element_type=jnp.float32) + nb_ref[...]
    for j in range(_NODE_NB):
        u = t[:, j * D:(j + 1) * D]
        m = jnp.max(u, axis=1, keepdims=True)
        e = jnp.exp(u - m)
        o_ref[:, j, :] = e / jnp.sum(e, axis=1, keepdims=True)


def _node_head(hd, node_w, node_b):
    grid = (MAXN // _NODE_NB,)
    return pl.pallas_call(
        _node_body,
        grid=grid,
        in_specs=[
            pl.BlockSpec((G, H), lambda i: (0, 0)),
            pl.BlockSpec((H, _NODE_COLS), lambda i: (0, i)),
            pl.BlockSpec((1, _NODE_COLS), lambda i: (0, i)),
        ],
        out_specs=pl.BlockSpec((G, _NODE_NB, D), lambda i: (0, i, 0)),
        out_shape=jax.ShapeDtypeStruct((G, MAXN, D), jnp.float32),
    )(hd, node_w, node_b.reshape(1, MAXN * D))


_ADJ_COLS = 1280


def _adj_body(hd_ref, ws_ref, bs_ref, o_ref):
    i = pl.program_id(0)
    t = jnp.dot(hd_ref[...], ws_ref[...], preferred_element_type=jnp.float32) + bs_ref[...]
    col = lax.broadcasted_iota(jnp.int32, (G, _ADJ_COLS), 1) + i * _ADJ_COLS
    diag = (col % (MAXN + 1)) == 0
    o_ref[...] = jnp.where(diag, -10.0, jnp.clip(t, -10.0, 10.0))


def _adj_head(hd, ws, bs):
    grid = (MAXN * MAXN // _ADJ_COLS,)
    return pl.pallas_call(
        _adj_body,
        grid=grid,
        in_specs=[
            pl.BlockSpec((G, H), lambda i: (0, 0)),
            pl.BlockSpec((H, _ADJ_COLS), lambda i: (0, i)),
            pl.BlockSpec((1, _ADJ_COLS), lambda i: (0, i)),
        ],
        out_specs=pl.BlockSpec((G, _ADJ_COLS), lambda i: (0, i)),
        out_shape=jax.ShapeDtypeStruct((G, MAXN * MAXN), jnp.float32),
    )(hd, ws, bs)


def kernel(x, edge_index, batch, params):
    p = params
    src = edge_index[0].astype(jnp.int32)
    dst = edge_index[1].astype(jnp.int32)
    src2 = jnp.concatenate([src, jnp.zeros((EPAD,), jnp.int32)]).reshape(ECP, CHUNK)
    pad_dst = N + (jnp.arange(EPAD, dtype=jnp.int32) % CHUNK)
    dst2 = jnp.concatenate([dst, pad_dst]).reshape(ECP, CHUNK)
    zero_init = jnp.zeros((NROW_TILE, D), jnp.float32)

    agg1 = _sc_scatter(x, src2, dst2, zero_init)
    h1 = _mlp(x, agg1, p['conv1_w1'], p['conv1_b1'].reshape(1, H),
              p['conv1_w2'], p['conv1_b2'].reshape(1, H))
    agg2 = _sc_scatter(h1, src2, dst2, zero_init)
    h2 = _mlp(h1, agg2, p['conv2_w1'], p['conv2_b1'].reshape(1, H),
              p['conv2_w2'], p['conv2_b2'].reshape(1, H))

    batch3 = batch.astype(jnp.int32).reshape(N // _MLP_R, 1, _MLP_R)
    sums, cnts = _pool(h2, batch3)

    eps = jax.random.normal(jax.random.key(42), (G, Z), dtype=jnp.float32)
    hd = _decode(sums, cnts, p, eps)

    node_features = _node_head(hd, p['node_w'], p['node_b'])

    ew3 = p['edge_w'].reshape(H, MAXN, MAXN)
    ws = ((ew3 + jnp.swapaxes(ew3, 1, 2)) * 0.5).reshape(H, MAXN * MAXN)
    eb2 = p['edge_b'].reshape(MAXN, MAXN)
    bs = ((eb2 + eb2.T) * 0.5).reshape(1, MAXN * MAXN)
    adj = _adj_head(hd, ws, bs).reshape(G, MAXN, MAXN)

    return adj, node_features


# speedup vs baseline: 2.6132x; 2.6132x over previous
import functools

import jax
import jax.numpy as jnp
from jax import lax
from jax.experimental import pallas as pl
from jax.experimental.pallas import tpu as pltpu
from jax.experimental.pallas import tpu_sc as plsc

N = 10000
D = 128
E = 320000
G = 64
H = 128
Z = 64
MAXN = 160

NC = 2
NS = 16
NW = NC * NS

CHUNK = 128
CPW = 80
ECP = CPW * NW
EPAD = ECP * CHUNK - E
NROW_TILE = 624
NREM = N - NROW_TILE * NS
NAGG = N + CHUNK


NBUF = 2
PCH = 40


def _sc_scatter_body(h_hbm, src_hbm, dst_hbm, zero_hbm, out_hbm,
                     src_v, dst_v, rows_v, gsems, ssems, agg):
    c = lax.axis_index("c")
    s = lax.axis_index("s")
    w = c * NS + s

    pltpu.sync_copy(zero_hbm.at[pl.ds(0, NROW_TILE)],
                    agg.at[pl.ds(s * NROW_TILE, NROW_TILE)])

    @pl.when(s == 0)
    def _():
        pltpu.sync_copy(zero_hbm.at[pl.ds(0, NREM + NAGG - N)],
                        agg.at[pl.ds(NS * NROW_TILE, NREM + NAGG - N)])

    plsc.subcore_barrier()

    for phase in range(CPW // PCH):
        base = w * CPW + phase * PCH
        pltpu.sync_copy(src_hbm.at[pl.ds(base, PCH)], src_v)
        pltpu.sync_copy(dst_hbm.at[pl.ds(base, PCH)], dst_v)

        for b in range(NBUF):
            pltpu.async_copy(h_hbm.at[src_v.at[b]], rows_v.at[b], gsems.at[b])

        @pl.loop(0, PCH - NBUF, step=NBUF)
        def _(j0):
            for b in range(NBUF):
                pltpu.make_async_copy(h_hbm.at[src_v.at[0]], rows_v.at[b],
                                      gsems.at[b]).wait()
                pltpu.async_copy(rows_v.at[b], agg.at[dst_v.at[j0 + b]],
                                 ssems.at[b], add=True)
            for b in range(NBUF):
                pltpu.make_async_copy(rows_v.at[b], agg.at[dst_v.at[0]],
                                      ssems.at[b]).wait()
                pltpu.async_copy(h_hbm.at[src_v.at[j0 + NBUF + b]],
                                 rows_v.at[b], gsems.at[b])

        for b in range(NBUF):
            pltpu.make_async_copy(h_hbm.at[src_v.at[0]], rows_v.at[b],
                                  gsems.at[b]).wait()
            pltpu.async_copy(rows_v.at[b], agg.at[dst_v.at[PCH - NBUF + b]],
                             ssems.at[b], add=True)
        for b in range(NBUF):
            pltpu.make_async_copy(rows_v.at[b], agg.at[dst_v.at[0]],
                                  ssems.at[b]).wait()

    plsc.subcore_barrier()
    pltpu.sync_copy(agg.at[pl.ds(s * NROW_TILE, NROW_TILE)],
                    out_hbm.at[pl.ds(c * N + s * NROW_TILE, NROW_TILE)])

    @pl.when(s == 0)
    def _():
        pltpu.sync_copy(agg.at[pl.ds(NS * NROW_TILE, NREM)],
                        out_hbm.at[pl.ds(c * N + NS * NROW_TILE, NREM)])


def _sc_scatter(h, src2, dst2, zero_init):
    mesh = plsc.VectorSubcoreMesh(core_axis_name="c", subcore_axis_name="s",
                                  num_cores=NC, num_subcores=NS)
    k = pl.kernel(
        _sc_scatter_body,
        out_type=jax.ShapeDtypeStruct((2 * N, D), jnp.float32),
        mesh=mesh,
        scratch_types=[
            pltpu.VMEM((PCH, CHUNK), jnp.int32),
            pltpu.VMEM((PCH, CHUNK), jnp.int32),
            pltpu.VMEM((NBUF, CHUNK, D), jnp.float32),
            pltpu.SemaphoreType.DMA((NBUF,)),
            pltpu.SemaphoreType.DMA((NBUF,)),
            pltpu.VMEM_SHARED((NAGG, D), jnp.float32),
        ],
    )
    return k(h, src2, dst2, zero_init)


def _mlp_body(h_ref, pa_ref, pb_ref, w1_ref, b1_ref, w2_ref, b2_ref, o_ref):
    x = h_ref[...] + pa_ref[...] + pb_ref[...]
    t = jnp.maximum(
        jnp.dot(x, w1_ref[...], preferred_element_type=jnp.float32) + b1_ref[...], 0.0)
    o_ref[...] = jnp.maximum(
        jnp.dot(t, w2_ref[...], preferred_element_type=jnp.float32) + b2_ref[...], 0.0)


_MLP_R = 1000


def _mlp(h, agg2, w1, b1, w2, b2):
    nb = N // _MLP_R
    return pl.pallas_call(
        _mlp_body,
        grid=(nb,),
        in_specs=[
            pl.BlockSpec((_MLP_R, D), lambda i: (i, 0)),
            pl.BlockSpec((_MLP_R, D), lambda i: (i, 0)),
            pl.BlockSpec((_MLP_R, D), lambda i, _nb=nb: (i + _nb, 0)),
            pl.BlockSpec((D, H), lambda i: (0, 0)),
            pl.BlockSpec((1, H), lambda i: (0, 0)),
            pl.BlockSpec((H, H), lambda i: (0, 0)),
            pl.BlockSpec((1, H), lambda i: (0, 0)),
        ],
        out_specs=pl.BlockSpec((_MLP_R, H), lambda i: (i, 0)),
        out_shape=jax.ShapeDtypeStruct((N, H), jnp.float32),
    )(h, agg2, agg2, w1, b1, w2, b2)


def _pool_body(h_ref, b_ref, sums_ref, cnts_ref):
    @pl.when(pl.program_id(0) == 0)
    def _():
        sums_ref[...] = jnp.zeros_like(sums_ref)
        cnts_ref[...] = jnp.zeros_like(cnts_ref)

    bb = b_ref[0, 0, :]
    onehot = (bb[:, None] == lax.broadcasted_iota(jnp.int32, (1, G), 1)
              ).astype(jnp.float32)
    sums_ref[...] += lax.dot_general(
        onehot, h_ref[...], (((0,), (0,)), ((), ())),
        preferred_element_type=jnp.float32)
    cnt = jnp.sum(onehot, axis=0)
    cnts_ref[...] += jnp.broadcast_to(cnt[:, None], (G, H))


def _pool(h2, batch3):
    nb = N // _MLP_R
    return pl.pallas_call(
        _pool_body,
        grid=(nb,),
        in_specs=[
            pl.BlockSpec((_MLP_R, H), lambda i: (i, 0)),
            pl.BlockSpec((1, 1, _MLP_R), lambda i: (i, 0, 0)),
        ],
        out_specs=[
            pl.BlockSpec((G, H), lambda i: (0, 0)),
            pl.BlockSpec((G, H), lambda i: (0, 0)),
        ],
        out_shape=[
            jax.ShapeDtypeStruct((G, H), jnp.float32),
            jax.ShapeDtypeStruct((G, H), jnp.float32),
        ],
    )(h2, batch3)


def _decode_body(sums_ref, cnts_ref, muw_ref, mub_ref, lvw_ref, lvb_ref,
                 eps_ref, dw1_ref, db1_ref, dw2_ref, db2_ref, hd_ref):
    pooled = sums_ref[...] / jnp.maximum(cnts_ref[...], 1.0)
    mu = jnp.dot(pooled, muw_ref[...], preferred_element_type=jnp.float32) + mub_ref[...]
    lv = jnp.clip(
        jnp.dot(pooled, lvw_ref[...], preferred_element_type=jnp.float32) + lvb_ref[...],
        -5.0, 5.0)
    z = mu + eps_ref[...] * jnp.exp(0.5 * lv)
    t = jnp.maximum(
        jnp.dot(z, dw1_ref[...], preferred_element_type=jnp.float32) + db1_ref[...], 0.0)
    hd_ref[...] = jnp.dot(t, dw2_ref[...], preferred_element_type=jnp.float32) + db2_ref[...]


def _decode(sums, cnts, p, eps):
    return pl.pallas_call(
        _decode_body,
        out_shape=jax.ShapeDtypeStruct((G, H), jnp.float32),
    )(sums, cnts, p['mu_w'], p['mu_b'].reshape(1, Z), p['lv_w'],
      p['lv_b'].reshape(1, Z), eps, p['dec_w1'], p['dec_b1'].reshape(1, H),
      p['dec_w2'], p['dec_b2'].reshape(1, H))


_NODE_NB = 8
_NODE_COLS = _NODE_NB * D


def _node_body(hd_ref, nw_ref, nb_ref, o_ref):
    t = jnp.dot(hd_ref[...], nw_ref[...], preferred_element_type=jnp.float32) + nb_ref[...]
    for j in range(_NODE_NB):
        u = t[:, j * D:(j + 1) * D]
        m = jnp.max(u, axis=1, keepdims=True)
        e = jnp.exp(u - m)
        o_ref[:, j, :] = e / jnp.sum(e, axis=1, keepdims=True)


def _node_head(hd, node_w, node_b):
    grid = (MAXN // _NODE_NB,)
    return pl.pallas_call(
        _node_body,
        grid=grid,
        in_specs=[
            pl.BlockSpec((G, H), lambda i: (0, 0)),
            pl.BlockSpec((H, _NODE_COLS), lambda i: (0, i)),
            pl.BlockSpec((1, _NODE_COLS), lambda i: (0, i)),
        ],
        out_specs=pl.BlockSpec((G, _NODE_NB, D), lambda i: (0, i, 0)),
        out_shape=jax.ShapeDtypeStruct((G, MAXN, D), jnp.float32),
    )(hd, node_w, node_b.reshape(1, MAXN * D))


_ADJ_COLS = 1280


def _adj_body(hd_ref, ws_ref, bs_ref, o_ref):
    i = pl.program_id(0)
    t = jnp.dot(hd_ref[...], ws_ref[...], preferred_element_type=jnp.float32) + bs_ref[...]
    col = lax.broadcasted_iota(jnp.int32, (G, _ADJ_COLS), 1) + i * _ADJ_COLS
    diag = (col % (MAXN + 1)) == 0
    o_ref[...] = jnp.where(diag, -10.0, jnp.clip(t, -10.0, 10.0))


def _adj_head(hd, ws, bs):
    grid = (MAXN * MAXN // _ADJ_COLS,)
    return pl.pallas_call(
        _adj_body,
        grid=grid,
        in_specs=[
            pl.BlockSpec((G, H), lambda i: (0, 0)),
            pl.BlockSpec((H, _ADJ_COLS), lambda i: (0, i)),
            pl.BlockSpec((1, _ADJ_COLS), lambda i: (0, i)),
        ],
        out_specs=pl.BlockSpec((G, _ADJ_COLS), lambda i: (0, i)),
        out_shape=jax.ShapeDtypeStruct((G, MAXN * MAXN), jnp.float32),
    )(hd, ws, bs)


def kernel(x, edge_index, batch, params):
    p = params
    src = edge_index[0].astype(jnp.int32)
    dst = edge_index[1].astype(jnp.int32)
    pad_src = jnp.arange(EPAD, dtype=jnp.int32) % N
    src2 = jnp.concatenate([src, pad_src]).reshape(ECP, CHUNK)
    pad_dst = N + (jnp.arange(EPAD, dtype=jnp.int32) % CHUNK)
    dst2 = jnp.concatenate([dst, pad_dst]).reshape(ECP, CHUNK)
    zero_init = jnp.zeros((NROW_TILE, D), jnp.float32)

    agg1 = _sc_scatter(x, src2, dst2, zero_init)
    h1 = _mlp(x, agg1, p['conv1_w1'], p['conv1_b1'].reshape(1, H),
              p['conv1_w2'], p['conv1_b2'].reshape(1, H))
    agg2 = _sc_scatter(h1, src2, dst2, zero_init)
    h2 = _mlp(h1, agg2, p['conv2_w1'], p['conv2_b1'].reshape(1, H),
              p['conv2_w2'], p['conv2_b2'].reshape(1, H))

    batch3 = batch.astype(jnp.int32).reshape(N // _MLP_R, 1, _MLP_R)
    sums, cnts = _pool(h2, batch3)

    eps = jax.random.normal(jax.random.key(42), (G, Z), dtype=jnp.float32)
    hd = _decode(sums, cnts, p, eps)

    node_features = _node_head(hd, p['node_w'], p['node_b'])

    ew3 = p['edge_w'].reshape(H, MAXN, MAXN)
    ws = ((ew3 + jnp.swapaxes(ew3, 1, 2)) * 0.5).reshape(H, MAXN * MAXN)
    eb2 = p['edge_b'].reshape(MAXN, MAXN)
    bs = ((eb2 + eb2.T) * 0.5).reshape(1, MAXN * MAXN)
    adj = _adj_head(hd, ws, bs).reshape(G, MAXN, MAXN)

    return adj, node_features


# speedup vs baseline: 2.9081x; 1.1128x over previous
import functools

import jax
import jax.numpy as jnp
from jax import lax
from jax.experimental import pallas as pl
from jax.experimental.pallas import tpu as pltpu
from jax.experimental.pallas import tpu_sc as plsc

N = 10000
D = 128
E = 320000
G = 64
H = 128
Z = 64
MAXN = 160

NC = 2
NS = 16
NW = NC * NS

CHUNK = 64
CPW = 160
ECP = CPW * NW
EPAD = ECP * CHUNK - E
NROW_TILE = 624
NREM = N - NROW_TILE * NS
NAGG = N + CHUNK


NBUF = 4
PCH = 40


def _sc_scatter_body(h_hbm, src_hbm, dst_hbm, zero_hbm, out_hbm,
                     src_v, dst_v, rows_v, gsems, ssems, agg):
    c = lax.axis_index("c")
    s = lax.axis_index("s")
    w = c * NS + s

    pltpu.sync_copy(zero_hbm.at[pl.ds(0, NROW_TILE)],
                    agg.at[pl.ds(s * NROW_TILE, NROW_TILE)])

    @pl.when(s == 0)
    def _():
        pltpu.sync_copy(zero_hbm.at[pl.ds(0, NREM + NAGG - N)],
                        agg.at[pl.ds(NS * NROW_TILE, NREM + NAGG - N)])

    plsc.subcore_barrier()

    for phase in range(CPW // PCH):
        base = w * CPW + phase * PCH
        pltpu.sync_copy(src_hbm.at[pl.ds(base, PCH)], src_v)
        pltpu.sync_copy(dst_hbm.at[pl.ds(base, PCH)], dst_v)

        for b in range(NBUF):
            pltpu.async_copy(h_hbm.at[src_v.at[b]], rows_v.at[b], gsems.at[b])

        @pl.loop(0, PCH - NBUF, step=NBUF)
        def _(j0):
            for b in range(NBUF):
                pltpu.make_async_copy(h_hbm.at[src_v.at[0]], rows_v.at[b],
                                      gsems.at[b]).wait()
                pltpu.async_copy(rows_v.at[b], agg.at[dst_v.at[j0 + b]],
                                 ssems.at[b], add=True)
            for b in range(NBUF):
                pltpu.make_async_copy(rows_v.at[b], agg.at[dst_v.at[0]],
                                      ssems.at[b]).wait()
                pltpu.async_copy(h_hbm.at[src_v.at[j0 + NBUF + b]],
                                 rows_v.at[b], gsems.at[b])

        for b in range(NBUF):
            pltpu.make_async_copy(h_hbm.at[src_v.at[0]], rows_v.at[b],
                                  gsems.at[b]).wait()
            pltpu.async_copy(rows_v.at[b], agg.at[dst_v.at[PCH - NBUF + b]],
                             ssems.at[b], add=True)
        for b in range(NBUF):
            pltpu.make_async_copy(rows_v.at[b], agg.at[dst_v.at[0]],
                                  ssems.at[b]).wait()

    plsc.subcore_barrier()
    pltpu.sync_copy(agg.at[pl.ds(s * NROW_TILE, NROW_TILE)],
                    out_hbm.at[pl.ds(c * N + s * NROW_TILE, NROW_TILE)])

    @pl.when(s == 0)
    def _():
        pltpu.sync_copy(agg.at[pl.ds(NS * NROW_TILE, NREM)],
                        out_hbm.at[pl.ds(c * N + NS * NROW_TILE, NREM)])


def _sc_scatter(h, src2, dst2, zero_init):
    mesh = plsc.VectorSubcoreMesh(core_axis_name="c", subcore_axis_name="s",
                                  num_cores=NC, num_subcores=NS)
    k = pl.kernel(
        _sc_scatter_body,
        out_type=jax.ShapeDtypeStruct((2 * N, D), jnp.float32),
        mesh=mesh,
        scratch_types=[
            pltpu.VMEM((PCH, CHUNK), jnp.int32),
            pltpu.VMEM((PCH, CHUNK), jnp.int32),
            pltpu.VMEM((NBUF, CHUNK, D), jnp.float32),
            pltpu.SemaphoreType.DMA((NBUF,)),
            pltpu.SemaphoreType.DMA((NBUF,)),
            pltpu.VMEM_SHARED((NAGG, D), jnp.float32),
        ],
    )
    return k(h, src2, dst2, zero_init)


def _mlp_body(h_ref, pa_ref, pb_ref, w1_ref, b1_ref, w2_ref, b2_ref, o_ref):
    x = h_ref[...] + pa_ref[...] + pb_ref[...]
    t = jnp.maximum(
        jnp.dot(x, w1_ref[...], preferred_element_type=jnp.float32) + b1_ref[...], 0.0)
    o_ref[...] = jnp.maximum(
        jnp.dot(t, w2_ref[...], preferred_element_type=jnp.float32) + b2_ref[...], 0.0)


_MLP_R = 1000


def _mlp(h, agg2, w1, b1, w2, b2):
    nb = N // _MLP_R
    return pl.pallas_call(
        _mlp_body,
        grid=(nb,),
        in_specs=[
            pl.BlockSpec((_MLP_R, D), lambda i: (i, 0)),
            pl.BlockSpec((_MLP_R, D), lambda i: (i, 0)),
            pl.BlockSpec((_MLP_R, D), lambda i, _nb=nb: (i + _nb, 0)),
            pl.BlockSpec((D, H), lambda i: (0, 0)),
            pl.BlockSpec((1, H), lambda i: (0, 0)),
            pl.BlockSpec((H, H), lambda i: (0, 0)),
            pl.BlockSpec((1, H), lambda i: (0, 0)),
        ],
        out_specs=pl.BlockSpec((_MLP_R, H), lambda i: (i, 0)),
        out_shape=jax.ShapeDtypeStruct((N, H), jnp.float32),
    )(h, agg2, agg2, w1, b1, w2, b2)


def _pool_body(h_ref, b_ref, sums_ref, cnts_ref):
    @pl.when(pl.program_id(0) == 0)
    def _():
        sums_ref[...] = jnp.zeros_like(sums_ref)
        cnts_ref[...] = jnp.zeros_like(cnts_ref)

    bb = b_ref[0, 0, :]
    onehot = (bb[:, None] == lax.broadcasted_iota(jnp.int32, (1, G), 1)
              ).astype(jnp.float32)
    sums_ref[...] += lax.dot_general(
        onehot, h_ref[...], (((0,), (0,)), ((), ())),
        preferred_element_type=jnp.float32)
    cnt = jnp.sum(onehot, axis=0)
    cnts_ref[...] += jnp.broadcast_to(cnt[:, None], (G, H))


def _pool(h2, batch3):
    nb = N // _MLP_R
    return pl.pallas_call(
        _pool_body,
        grid=(nb,),
        in_specs=[
            pl.BlockSpec((_MLP_R, H), lambda i: (i, 0)),
            pl.BlockSpec((1, 1, _MLP_R), lambda i: (i, 0, 0)),
        ],
        out_specs=[
            pl.BlockSpec((G, H), lambda i: (0, 0)),
            pl.BlockSpec((G, H), lambda i: (0, 0)),
        ],
        out_shape=[
            jax.ShapeDtypeStruct((G, H), jnp.float32),
            jax.ShapeDtypeStruct((G, H), jnp.float32),
        ],
    )(h2, batch3)


def _decode_body(sums_ref, cnts_ref, muw_ref, mub_ref, lvw_ref, lvb_ref,
                 eps_ref, dw1_ref, db1_ref, dw2_ref, db2_ref, hd_ref):
    pooled = sums_ref[...] / jnp.maximum(cnts_ref[...], 1.0)
    mu = jnp.dot(pooled, muw_ref[...], preferred_element_type=jnp.float32) + mub_ref[...]
    lv = jnp.clip(
        jnp.dot(pooled, lvw_ref[...], preferred_element_type=jnp.float32) + lvb_ref[...],
        -5.0, 5.0)
    z = mu + eps_ref[...] * jnp.exp(0.5 * lv)
    t = jnp.maximum(
        jnp.dot(z, dw1_ref[...], preferred_element_type=jnp.float32) + db1_ref[...], 0.0)
    hd_ref[...] = jnp.dot(t, dw2_ref[...], preferred_element_type=jnp.float32) + db2_ref[...]


def _decode(sums, cnts, p, eps):
    return pl.pallas_call(
        _decode_body,
        out_shape=jax.ShapeDtypeStruct((G, H), jnp.float32),
    )(sums, cnts, p['mu_w'], p['mu_b'].reshape(1, Z), p['lv_w'],
      p['lv_b'].reshape(1, Z), eps, p['dec_w1'], p['dec_b1'].reshape(1, H),
      p['dec_w2'], p['dec_b2'].reshape(1, H))


_NODE_NB = 8
_NODE_COLS = _NODE_NB * D


def _node_body(hd_ref, nw_ref, nb_ref, o_ref):
    t = jnp.dot(hd_ref[...], nw_ref[...], preferred_element_type=jnp.float32) + nb_ref[...]
    for j in range(_NODE_NB):
        u = t[:, j * D:(j + 1) * D]
        m = jnp.max(u, axis=1, keepdims=True)
        e = jnp.exp(u - m)
        o_ref[:, j, :] = e / jnp.sum(e, axis=1, keepdims=True)


def _node_head(hd, node_w, node_b):
    grid = (MAXN // _NODE_NB,)
    return pl.pallas_call(
        _node_body,
        grid=grid,
        in_specs=[
            pl.BlockSpec((G, H), lambda i: (0, 0)),
            pl.BlockSpec((H, _NODE_COLS), lambda i: (0, i)),
            pl.BlockSpec((1, _NODE_COLS), lambda i: (0, i)),
        ],
        out_specs=pl.BlockSpec((G, _NODE_NB, D), lambda i: (0, i, 0)),
        out_shape=jax.ShapeDtypeStruct((G, MAXN, D), jnp.float32),
    )(hd, node_w, node_b.reshape(1, MAXN * D))


_ADJ_COLS = 1280


def _adj_body(hd_ref, ws_ref, bs_ref, o_ref):
    i = pl.program_id(0)
    t = jnp.dot(hd_ref[...], ws_ref[...], preferred_element_type=jnp.float32) + bs_ref[...]
    col = lax.broadcasted_iota(jnp.int32, (G, _ADJ_COLS), 1) + i * _ADJ_COLS
    diag = (col % (MAXN + 1)) == 0
    o_ref[...] = jnp.where(diag, -10.0, jnp.clip(t, -10.0, 10.0))


def _adj_head(hd, ws, bs):
    grid = (MAXN * MAXN // _ADJ_COLS,)
    return pl.pallas_call(
        _adj_body,
        grid=grid,
        in_specs=[
            pl.BlockSpec((G, H), lambda i: (0, 0)),
            pl.BlockSpec((H, _ADJ_COLS), lambda i: (0, i)),
            pl.BlockSpec((1, _ADJ_COLS), lambda i: (0, i)),
        ],
        out_specs=pl.BlockSpec((G, _ADJ_COLS), lambda i: (0, i)),
        out_shape=jax.ShapeDtypeStruct((G, MAXN * MAXN), jnp.float32),
    )(hd, ws, bs)


def kernel(x, edge_index, batch, params):
    p = params
    src = edge_index[0].astype(jnp.int32)
    dst = edge_index[1].astype(jnp.int32)
    pad_src = jnp.arange(EPAD, dtype=jnp.int32) % N
    src2 = jnp.concatenate([src, pad_src]).reshape(ECP, CHUNK)
    pad_dst = N + (jnp.arange(EPAD, dtype=jnp.int32) % CHUNK)
    dst2 = jnp.concatenate([dst, pad_dst]).reshape(ECP, CHUNK)
    zero_init = jnp.zeros((NROW_TILE, D), jnp.float32)

    agg1 = _sc_scatter(x, src2, dst2, zero_init)
    h1 = _mlp(x, agg1, p['conv1_w1'], p['conv1_b1'].reshape(1, H),
              p['conv1_w2'], p['conv1_b2'].reshape(1, H))
    agg2 = _sc_scatter(h1, src2, dst2, zero_init)
    h2 = _mlp(h1, agg2, p['conv2_w1'], p['conv2_b1'].reshape(1, H),
              p['conv2_w2'], p['conv2_b2'].reshape(1, H))

    batch3 = batch.astype(jnp.int32).reshape(N // _MLP_R, 1, _MLP_R)
    sums, cnts = _pool(h2, batch3)

    eps = jax.random.normal(jax.random.key(42), (G, Z), dtype=jnp.float32)
    hd = _decode(sums, cnts, p, eps)

    node_features = _node_head(hd, p['node_w'], p['node_b'])

    ew3 = p['edge_w'].reshape(H, MAXN, MAXN)
    ws = ((ew3 + jnp.swapaxes(ew3, 1, 2)) * 0.5).reshape(H, MAXN * MAXN)
    eb2 = p['edge_b'].reshape(MAXN, MAXN)
    bs = ((eb2 + eb2.T) * 0.5).reshape(1, MAXN * MAXN)
    adj = _adj_head(hd, ws, bs).reshape(G, MAXN, MAXN)

    return adj, node_features
